# Initial kernel scaffold; baseline (speedup 1.0000x reference)
import jax
import jax.numpy as jnp
from jax.experimental import pallas as pl


def kernel(x, edge_index, gcn_W, gcn_b, W1, b1, W2, b2, Wout, bout):
    raise NotImplementedError("write your pallas kernel here")


# speedup vs baseline: 13056.6134x; 13056.6134x over previous
import functools

import jax
import jax.numpy as jnp
from jax import lax
from jax.experimental import pallas as pl
from jax.experimental.pallas import tpu as pltpu
from jax.experimental.pallas import tpu_sc as plsc

_NEQ = 10000
_MU = 500
_NPAD = 10240
_NPT = _NPAD // 16
_E = 160000
_EPAD = 163840
_ECH = 128
_CHUNKS = _EPAD // 16 // _ECH
_PAD_SLOT = 10016

_KP = 10752
_KB = 1536
_NKB = _KP // _KB
_NOUTP = 10624


def _rsqrt16(x):
    y = 1.0 / x
    for _ in range(24):
        y = y * (1.5 - 0.5 * x * y * y)
    return y


def _gcn_sc_body(rows_hbm, cols_hbm, v_hbm, wb_hbm, out_hbm,
                 ridx, cidx, msg, ones, wb_v,
                 cnt_loc, v_loc, g_loc, dinv_loc, s_loc, out_loc,
                 sh_cnt, sh_g, sh_s, sem):
    c = lax.axis_index("c")
    sid = lax.axis_index("s")

    @pl.when(c == 0)
    def _body():
        nb = sid * _NPT
        eb = sid * _CHUNKS

        def z_body(k, carry):
            out_loc[pl.ds(k * 16, 16)] = jnp.zeros((16,), jnp.float32)
            return carry
        lax.fori_loop(0, _NPT // 16, z_body, None)
        for k in range(_ECH // 16):
            ones[pl.ds(k * 16, 16)] = jnp.ones((16,), jnp.float32)
        pltpu.sync_copy(out_loc, sh_cnt.at[pl.ds(nb, _NPT)])
        pltpu.sync_copy(out_loc, sh_s.at[pl.ds(nb, _NPT)])
        pltpu.sync_copy(wb_hbm, wb_v)
        pltpu.sync_copy(cols_hbm.at[pl.ds(eb, _CHUNKS)], cidx)
        pltpu.sync_copy(rows_hbm.at[pl.ds(eb, _CHUNKS)], ridx)
        plsc.subcore_barrier()

        def cnt_body(j, carry):
            pltpu.sync_copy(ones, sh_cnt.at[cidx.at[j]], add=True)
            return carry
        lax.fori_loop(0, _CHUNKS, cnt_body, None)
        plsc.subcore_barrier()

        pltpu.sync_copy(sh_cnt.at[pl.ds(nb, _NPT)], cnt_loc)
        pltpu.sync_copy(v_hbm.at[pl.ds(nb, _NPT)], v_loc)
        wvec = wb_v[pl.ds(0, 16)]

        def ew_body(k, carry):
            cnt16 = cnt_loc[pl.ds(k * 16, 16)]
            v16 = v_loc[pl.ds(k * 16, 16)]
            deg = cnt16 * 128.0 + 1.0
            y = _rsqrt16(deg)
            dinv_loc[pl.ds(k * 16, 16)] = y
            g_loc[pl.ds(k * 16, 16)] = y * v16 * wvec
            return carry
        lax.fori_loop(0, _NPT // 16, ew_body, None)
        pltpu.sync_copy(g_loc, sh_g.at[pl.ds(nb, _NPT)])
        plsc.subcore_barrier()

        def msg_body(j, carry):
            pltpu.async_copy(sh_g.at[ridx.at[j]], msg.at[j], sem).wait()
            pltpu.sync_copy(msg.at[j], sh_s.at[cidx.at[j]], add=True)
            return carry
        lax.fori_loop(0, _CHUNKS, msg_body, None)
        plsc.subcore_barrier()

        pltpu.sync_copy(sh_s.at[pl.ds(nb, _NPT)], s_loc)
        bvec = wb_v[pl.ds(16, 16)]

        def out_body(k, carry):
            s16 = s_loc[pl.ds(k * 16, 16)]
            d16 = dinv_loc[pl.ds(k * 16, 16)]
            v16 = v_loc[pl.ds(k * 16, 16)]
            o = 128.0 * d16 * s16 + d16 * d16 * v16 * wvec + bvec
            out_loc[pl.ds(k * 16, 16)] = jnp.maximum(o, 0.0)
            return carry
        lax.fori_loop(0, _NPT // 16, out_body, None)
        pltpu.sync_copy(out_loc, out_hbm.at[pl.ds(nb, _NPT)])


def _gcn_sc(rows_p, cols_p, v_p, wb):
    mesh = plsc.VectorSubcoreMesh(core_axis_name="c", subcore_axis_name="s")
    return pl.kernel(
        _gcn_sc_body,
        out_type=jax.ShapeDtypeStruct((_NPAD,), jnp.float32),
        mesh=mesh,
        scratch_types=[
            pltpu.VMEM((_CHUNKS, _ECH), jnp.int32),
            pltpu.VMEM((_CHUNKS, _ECH), jnp.int32),
            pltpu.VMEM((_CHUNKS, _ECH), jnp.float32),
            pltpu.VMEM((_ECH,), jnp.float32),
            pltpu.VMEM((32,), jnp.float32),
            pltpu.VMEM((_NPT,), jnp.float32),
            pltpu.VMEM((_NPT,), jnp.float32),
            pltpu.VMEM((_NPT,), jnp.float32),
            pltpu.VMEM((_NPT,), jnp.float32),
            pltpu.VMEM((_NPT,), jnp.float32),
            pltpu.VMEM((_NPT,), jnp.float32),
            pltpu.VMEM_SHARED((_NPAD,), jnp.float32),
            pltpu.VMEM_SHARED((_NPAD,), jnp.float32),
            pltpu.VMEM_SHARED((_NPAD,), jnp.float32),
            pltpu.SemaphoreType.DMA,
        ],
    )(rows_p, cols_p, v_p, wb)


def _mlp_tc_body(x_ref, os_ref, w1_ref, w2_ref, wout_ref,
                 b1_ref, b2_ref, bout_ref, wb_ref, o_ref, acc_ref):
    i = pl.program_id(0)
    gcol = lax.broadcasted_iota(jnp.int32, (128, _KB), 1) + i * _KB
    brow = lax.broadcasted_iota(jnp.int32, (128, _KB), 0)
    xb = x_ref[...]
    w = wb_ref[0, 0]
    gb = wb_ref[0, 1]
    base = jnp.where(gcol < _NEQ, jnp.maximum(w * xb + gb, 0.0), xb)
    flat = gcol * 128 + brow
    sub = jnp.where(flat[:, :128] < _NEQ, os_ref[...], base[:, :128])
    xcat = jnp.concatenate([sub, base[:, 128:]], axis=1)

    @pl.when(i == 0)
    def _init():
        acc_ref[...] = jnp.zeros_like(acc_ref)

    acc_ref[...] += lax.dot_general(
        xcat, w1_ref[...], (((1,), (1,)), ((), ())),
        preferred_element_type=jnp.float32)

    @pl.when(i == _NKB - 1)
    def _epilogue():
        h1 = jnp.maximum(acc_ref[...] + b1_ref[...], 0.0)
        h2 = jnp.maximum(
            lax.dot_general(h1, w2_ref[...], (((1,), (1,)), ((), ())),
                            preferred_element_type=jnp.float32) + b2_ref[...],
            0.0)
        o_ref[...] = lax.dot_general(
            h2, wout_ref[...], (((1,), (1,)), ((), ())),
            preferred_element_type=jnp.float32) + bout_ref[...]


def _mlp_tc(x_p, os_mat, W1_p, W2, Wout_p, b1, b2, bout_p, wb2):
    return pl.pallas_call(
        _mlp_tc_body,
        grid=(_NKB,),
        in_specs=[
            pl.BlockSpec((128, _KB), lambda i: (0, i)),
            pl.BlockSpec((128, 128), lambda i: (0, 0)),
            pl.BlockSpec((512, _KB), lambda i: (0, i)),
            pl.BlockSpec((256, 512), lambda i: (0, 0)),
            pl.BlockSpec((_NOUTP, 256), lambda i: (0, 0)),
            pl.BlockSpec((1, 512), lambda i: (0, 0)),
            pl.BlockSpec((1, 256), lambda i: (0, 0)),
            pl.BlockSpec((1, _NOUTP), lambda i: (0, 0)),
            pl.BlockSpec((1, 2), lambda i: (0, 0)),
        ],
        out_specs=pl.BlockSpec((128, _NOUTP), lambda i: (0, 0)),
        out_shape=jax.ShapeDtypeStruct((128, _NOUTP), jnp.float32),
        scratch_shapes=[pltpu.VMEM((128, 512), jnp.float32)],
    )(x_p, os_mat, W1_p, W2, Wout_p, b1, b2, bout_p, wb2)


def kernel(x, edge_index, gcn_W, gcn_b, W1, b1, W2, b2, Wout, bout):
    w = gcn_W[0, 0]
    row = edge_index[0]
    col = edge_index[1]

    rows_p = jnp.concatenate(
        [row, jnp.zeros((_EPAD - _E,), jnp.int32)]).reshape(_EPAD // _ECH, _ECH)
    cols_p = jnp.concatenate(
        [col, jnp.full((_EPAD - _E,), _PAD_SLOT, jnp.int32)]
    ).reshape(_EPAD // _ECH, _ECH)
    v_p = x[:, : _NPAD // 128].T.reshape(-1)
    wb = jnp.concatenate([jnp.full((16,), w, jnp.float32),
                          jnp.full((16,), gcn_b[0], jnp.float32)])

    out_small = _gcn_sc(rows_p, cols_p, v_p, wb)

    os_mat = out_small.reshape(_NPAD // 128, 128).T

    x_p = jnp.pad(x, ((0, 0), (0, _KP - 10500)))
    W1_p = jnp.pad(W1, ((0, 0), (0, _KP - 10500)))
    Wout_p = jnp.pad(Wout, ((0, _NOUTP - 10500), (0, 0)))
    bout_p = jnp.pad(bout, (0, _NOUTP - 10500)).reshape(1, _NOUTP)
    wb2 = jnp.stack([w, gcn_b[0]]).reshape(1, 2)

    o = _mlp_tc(x_p, os_mat, W1_p, W2, Wout_p,
                b1.reshape(1, 512), b2.reshape(1, 256), bout_p, wb2)

    o = o[:, :10500]
    return (o[:, :_MU], o[:, _MU:])


# speedup vs baseline: 17286.7801x; 1.3240x over previous
import functools

import jax
import jax.numpy as jnp
from jax import lax
from jax.experimental import pallas as pl
from jax.experimental.pallas import tpu as pltpu
from jax.experimental.pallas import tpu_sc as plsc

_NEQ = 10000
_MU = 500
_NPAD = 10240
_NPT = _NPAD // 16
_E = 160000
_EPAD = 163840
_ECH = 128
_CHUNKS = _EPAD // 16 // _ECH
_WAVE = 8
_PAD_SLOT = 10016

_NIN = 10500
_FB = 128
_NFB = 512 // _FB


def _rsqrt16(x):
    y = 1.0 / x
    for _ in range(24):
        y = y * (1.5 - 0.5 * x * y * y)
    return y


def _gcn_sc_body(rows_hbm, cols_hbm, v_hbm, wb_hbm, out_hbm,
                 ridx, cidx, msg, ones, wb_v,
                 cnt_loc, v_loc, g_loc, dinv_loc, s_loc, out_loc,
                 sh_cnt, sh_g, sh_s, sem, sem2):
    c = lax.axis_index("c")
    sid = lax.axis_index("s")

    @pl.when(c == 0)
    def _body():
        nb = sid * _NPT
        eb = sid * _CHUNKS

        def z_body(k, carry):
            out_loc[pl.ds(k * 16, 16)] = jnp.zeros((16,), jnp.float32)
            return carry
        lax.fori_loop(0, _NPT // 16, z_body, None)
        for k in range(_ECH // 16):
            ones[pl.ds(k * 16, 16)] = jnp.ones((16,), jnp.float32)
        pltpu.sync_copy(out_loc, sh_cnt.at[pl.ds(nb, _NPT)])
        pltpu.sync_copy(out_loc, sh_s.at[pl.ds(nb, _NPT)])
        pltpu.sync_copy(wb_hbm, wb_v)
        pltpu.sync_copy(cols_hbm.at[pl.ds(eb, _CHUNKS)], cidx)
        pltpu.sync_copy(rows_hbm.at[pl.ds(eb, _CHUNKS)], ridx)
        plsc.subcore_barrier()

        def cnt_wave(wv, carry):
            for b in range(_WAVE):
                j = wv * _WAVE + b
                pltpu.async_copy(ones, sh_cnt.at[cidx.at[j]], sem, add=True)
            for b in range(_WAVE):
                j = wv * _WAVE + b
                pltpu.make_async_copy(ones, sh_cnt.at[cidx.at[j]], sem).wait()
            return carry
        lax.fori_loop(0, _CHUNKS // _WAVE, cnt_wave, None)
        plsc.subcore_barrier()

        pltpu.sync_copy(sh_cnt.at[pl.ds(nb, _NPT)], cnt_loc)
        pltpu.sync_copy(v_hbm.at[pl.ds(nb, _NPT)], v_loc)
        wvec = wb_v[pl.ds(0, 16)]

        def ew_body(k, carry):
            cnt16 = cnt_loc[pl.ds(k * 16, 16)]
            v16 = v_loc[pl.ds(k * 16, 16)]
            deg = cnt16 * 128.0 + 1.0
            y = _rsqrt16(deg)
            dinv_loc[pl.ds(k * 16, 16)] = y
            g_loc[pl.ds(k * 16, 16)] = y * v16 * wvec
            return carry
        lax.fori_loop(0, _NPT // 16, ew_body, None)
        pltpu.sync_copy(g_loc, sh_g.at[pl.ds(nb, _NPT)])
        plsc.subcore_barrier()

        def msg_wave(wv, carry):
            for b in range(_WAVE):
                j = wv * _WAVE + b
                pltpu.async_copy(sh_g.at[ridx.at[j]], msg.at[j], sem2)
            for b in range(_WAVE):
                j = wv * _WAVE + b
                pltpu.make_async_copy(sh_g.at[ridx.at[j]], msg.at[j], sem2).wait()
            for b in range(_WAVE):
                j = wv * _WAVE + b
                pltpu.async_copy(msg.at[j], sh_s.at[cidx.at[j]], sem, add=True)
            for b in range(_WAVE):
                j = wv * _WAVE + b
                pltpu.make_async_copy(msg.at[j], sh_s.at[cidx.at[j]], sem).wait()
            return carry
        lax.fori_loop(0, _CHUNKS // _WAVE, msg_wave, None)
        plsc.subcore_barrier()

        pltpu.sync_copy(sh_s.at[pl.ds(nb, _NPT)], s_loc)
        bvec = wb_v[pl.ds(16, 16)]

        def out_body(k, carry):
            s16 = s_loc[pl.ds(k * 16, 16)]
            d16 = dinv_loc[pl.ds(k * 16, 16)]
            v16 = v_loc[pl.ds(k * 16, 16)]
            o = 128.0 * d16 * s16 + d16 * d16 * v16 * wvec + bvec
            out_loc[pl.ds(k * 16, 16)] = jnp.maximum(o, 0.0)
            return carry
        lax.fori_loop(0, _NPT // 16, out_body, None)
        pltpu.sync_copy(out_loc, out_hbm.at[pl.ds(nb, _NPT)])


def _gcn_sc(rows_p, cols_p, v_p, wb):
    mesh = plsc.VectorSubcoreMesh(core_axis_name="c", subcore_axis_name="s")
    return pl.kernel(
        _gcn_sc_body,
        out_type=jax.ShapeDtypeStruct((_NPAD,), jnp.float32),
        mesh=mesh,
        scratch_types=[
            pltpu.VMEM((_CHUNKS, _ECH), jnp.int32),
            pltpu.VMEM((_CHUNKS, _ECH), jnp.int32),
            pltpu.VMEM((_CHUNKS, _ECH), jnp.float32),
            pltpu.VMEM((_ECH,), jnp.float32),
            pltpu.VMEM((32,), jnp.float32),
            pltpu.VMEM((_NPT,), jnp.float32),
            pltpu.VMEM((_NPT,), jnp.float32),
            pltpu.VMEM((_NPT,), jnp.float32),
            pltpu.VMEM((_NPT,), jnp.float32),
            pltpu.VMEM((_NPT,), jnp.float32),
            pltpu.VMEM((_NPT,), jnp.float32),
            pltpu.VMEM_SHARED((_NPAD,), jnp.float32),
            pltpu.VMEM_SHARED((_NPAD,), jnp.float32),
            pltpu.VMEM_SHARED((_NPAD,), jnp.float32),
            pltpu.SemaphoreType.DMA,
            pltpu.SemaphoreType.DMA,
        ],
    )(rows_p, cols_p, v_p, wb)


def _mlp_tc_body(x_ref, os_ref, w1_ref, w2_ref, wout_ref,
                 b1_ref, b2_ref, bout_ref, wb_ref, o_ref, xcat_s, h1_s):
    i = pl.program_id(0)

    @pl.when(i == 0)
    def _build():
        gcol = lax.broadcasted_iota(jnp.int32, (128, _NIN), 1)
        xb = x_ref[...]
        w = wb_ref[0, 0]
        gb = wb_ref[0, 1]
        base = jnp.where(gcol < _NEQ, jnp.maximum(w * xb + gb, 0.0), xb)
        brow = lax.broadcasted_iota(jnp.int32, (128, 128), 0)
        flat = gcol[:, :128] * 128 + brow
        sub = jnp.where(flat < _NEQ, os_ref[...], base[:, :128])
        xcat_s[...] = jnp.concatenate([sub, base[:, 128:]], axis=1)

    h1_s[:, pl.ds(i * _FB, _FB)] = lax.dot_general(
        xcat_s[...], w1_ref[...], (((1,), (1,)), ((), ())),
        preferred_element_type=jnp.float32)

    @pl.when(i == _NFB - 1)
    def _epilogue():
        h1 = jnp.maximum(h1_s[...] + b1_ref[...], 0.0)
        h2 = jnp.maximum(
            lax.dot_general(h1, w2_ref[...], (((1,), (1,)), ((), ())),
                            preferred_element_type=jnp.float32) + b2_ref[...],
            0.0)
        o_ref[...] = lax.dot_general(
            h2, wout_ref[...], (((1,), (1,)), ((), ())),
            preferred_element_type=jnp.float32) + bout_ref[...]


def _mlp_tc(x, os_mat, W1, W2, Wout, b1, b2, bout, wb2):
    return pl.pallas_call(
        _mlp_tc_body,
        grid=(_NFB,),
        in_specs=[
            pl.BlockSpec((128, _NIN), lambda i: (0, 0)),
            pl.BlockSpec((128, 128), lambda i: (0, 0)),
            pl.BlockSpec((_FB, _NIN), lambda i: (i, 0)),
            pl.BlockSpec((256, 512), lambda i: (0, 0)),
            pl.BlockSpec((_NIN, 256), lambda i: (0, 0)),
            pl.BlockSpec((1, 512), lambda i: (0, 0)),
            pl.BlockSpec((1, 256), lambda i: (0, 0)),
            pl.BlockSpec((1, _NIN), lambda i: (0, 0)),
            pl.BlockSpec((1, 2), lambda i: (0, 0)),
        ],
        out_specs=pl.BlockSpec((128, _NIN), lambda i: (0, 0)),
        out_shape=jax.ShapeDtypeStruct((128, _NIN), jnp.float32),
        scratch_shapes=[pltpu.VMEM((128, _NIN), jnp.float32),
                        pltpu.VMEM((128, 512), jnp.float32)],
    )(x, os_mat, W1, W2, Wout, b1, b2, bout, wb2)


def kernel(x, edge_index, gcn_W, gcn_b, W1, b1, W2, b2, Wout, bout):
    w = gcn_W[0, 0]
    row = edge_index[0]
    col = edge_index[1]

    rows_p = jnp.concatenate(
        [row, jnp.zeros((_EPAD - _E,), jnp.int32)]).reshape(_EPAD // _ECH, _ECH)
    cols_p = jnp.concatenate(
        [col, jnp.full((_EPAD - _E,), _PAD_SLOT, jnp.int32)]
    ).reshape(_EPAD // _ECH, _ECH)
    v_p = x[:, : _NPAD // 128].T.reshape(-1)
    wb = jnp.concatenate([jnp.full((16,), w, jnp.float32),
                          jnp.full((16,), gcn_b[0], jnp.float32)])

    out_small = _gcn_sc(rows_p, cols_p, v_p, wb)

    os_mat = jnp.pad(out_small.reshape(_NPAD // 128, 128).T, ((0, 0), (0, 48)))

    wb2 = jnp.stack([w, gcn_b[0]]).reshape(1, 2)
    o = _mlp_tc(x, os_mat, W1, W2, Wout,
                b1.reshape(1, 512), b2.reshape(1, 256),
                bout.reshape(1, _NIN), wb2)

    return (o[:, :_MU], o[:, _MU:])


# speedup vs baseline: 24390.4213x; 1.4109x over previous
import jax
import jax.numpy as jnp
from jax import lax
from jax.experimental import pallas as pl
from jax.experimental.pallas import tpu as pltpu
from jax.experimental.pallas import tpu_sc as plsc

_NEQ = 10000
_MU = 500
_NPAD = 10240
_NPT = _NPAD // 16
_E = 160000
_EPAD = 163840
_ECH = 128
_CHUNKS = _EPAD // 16 // _ECH
_WAVE = 8
_PAD_SLOT = 10016

_NIN = 10500
_FB = 128
_NFB = 512 // _FB


def _rsqrt16(x):
    y = 1.0 / x
    for _ in range(24):
        y = y * (1.5 - 0.5 * x * y * y)
    return y


def _gcn_sc_body(ei_hbm, v_hbm, wb_hbm, out_hbm,
                 ridx, cidx, msg, ones, wb_v,
                 cnt_loc, v_loc, g_loc, dinv_loc, s_loc, out_loc,
                 sh_cnt, sh_g, sh_s, sem, sem2):
    c = lax.axis_index("c")
    sid = lax.axis_index("s")

    @pl.when(c == 0)
    def _body():
        nb = sid * _NPT
        eb = sid * _CHUNKS

        def z_body(k, carry):
            out_loc[pl.ds(k * 16, 16)] = jnp.zeros((16,), jnp.float32)
            return carry
        lax.fori_loop(0, _NPT // 16, z_body, None)
        for k in range(_ECH // 16):
            ones[pl.ds(k * 16, 16)] = jnp.ones((16,), jnp.float32)
        pltpu.sync_copy(out_loc, sh_cnt.at[pl.ds(nb, _NPT)])
        pltpu.sync_copy(out_loc, sh_s.at[pl.ds(nb, _NPT)])
        pltpu.sync_copy(wb_hbm, wb_v)
        pltpu.sync_copy(ei_hbm.at[0, pl.ds(eb, _CHUNKS)], ridx)
        pltpu.sync_copy(ei_hbm.at[1, pl.ds(eb, _CHUNKS)], cidx)
        plsc.subcore_barrier()

        def cnt_wave(wv, carry):
            for b in range(_WAVE):
                j = wv * _WAVE + b
                pltpu.async_copy(ones, sh_cnt.at[cidx.at[j]], sem, add=True)
            for b in range(_WAVE):
                j = wv * _WAVE + b
                pltpu.make_async_copy(ones, sh_cnt.at[cidx.at[j]], sem).wait()
            return carry
        lax.fori_loop(0, _CHUNKS // _WAVE, cnt_wave, None)
        plsc.subcore_barrier()

        pltpu.sync_copy(sh_cnt.at[pl.ds(nb, _NPT)], cnt_loc)
        pltpu.sync_copy(v_hbm.at[pl.ds(nb, _NPT)], v_loc)
        wvec = wb_v[pl.ds(0, 16)]

        def ew_body(k, carry):
            cnt16 = cnt_loc[pl.ds(k * 16, 16)]
            v16 = v_loc[pl.ds(k * 16, 16)]
            deg = cnt16 * 128.0 + 1.0
            y = _rsqrt16(deg)
            dinv_loc[pl.ds(k * 16, 16)] = y
            g_loc[pl.ds(k * 16, 16)] = y * v16 * wvec
            return carry
        lax.fori_loop(0, _NPT // 16, ew_body, None)
        pltpu.sync_copy(g_loc, sh_g.at[pl.ds(nb, _NPT)])
        plsc.subcore_barrier()

        def msg_wave(wv, carry):
            for b in range(_WAVE):
                j = wv * _WAVE + b
                pltpu.async_copy(sh_g.at[ridx.at[j]], msg.at[j], sem2)
            for b in range(_WAVE):
                j = wv * _WAVE + b
                pltpu.make_async_copy(sh_g.at[ridx.at[j]], msg.at[j], sem2).wait()
            for b in range(_WAVE):
                j = wv * _WAVE + b
                pltpu.async_copy(msg.at[j], sh_s.at[cidx.at[j]], sem, add=True)
            for b in range(_WAVE):
                j = wv * _WAVE + b
                pltpu.make_async_copy(msg.at[j], sh_s.at[cidx.at[j]], sem).wait()
            return carry
        lax.fori_loop(0, _CHUNKS // _WAVE, msg_wave, None)
        plsc.subcore_barrier()

        pltpu.sync_copy(sh_s.at[pl.ds(nb, _NPT)], s_loc)
        bvec = wb_v[pl.ds(16, 16)]

        def out_body(k, carry):
            s16 = s_loc[pl.ds(k * 16, 16)]
            d16 = dinv_loc[pl.ds(k * 16, 16)]
            v16 = v_loc[pl.ds(k * 16, 16)]
            o = 128.0 * d16 * s16 + d16 * d16 * v16 * wvec + bvec
            out_loc[pl.ds(k * 16, 16)] = jnp.maximum(o, 0.0)
            return carry
        lax.fori_loop(0, _NPT // 16, out_body, None)
        pltpu.sync_copy(out_loc, out_hbm.at[pl.ds(nb, _NPT)])


def _gcn_sc(ei_p, v_p, wb):
    mesh = plsc.VectorSubcoreMesh(core_axis_name="c", subcore_axis_name="s")
    return pl.kernel(
        _gcn_sc_body,
        out_type=jax.ShapeDtypeStruct((_NPAD,), jnp.float32),
        mesh=mesh,
        scratch_types=[
            pltpu.VMEM((_CHUNKS, _ECH), jnp.int32),
            pltpu.VMEM((_CHUNKS, _ECH), jnp.int32),
            pltpu.VMEM((_CHUNKS, _ECH), jnp.float32),
            pltpu.VMEM((_ECH,), jnp.float32),
            pltpu.VMEM((32,), jnp.float32),
            pltpu.VMEM((_NPT,), jnp.float32),
            pltpu.VMEM((_NPT,), jnp.float32),
            pltpu.VMEM((_NPT,), jnp.float32),
            pltpu.VMEM((_NPT,), jnp.float32),
            pltpu.VMEM((_NPT,), jnp.float32),
            pltpu.VMEM((_NPT,), jnp.float32),
            pltpu.VMEM_SHARED((_NPAD,), jnp.float32),
            pltpu.VMEM_SHARED((_NPAD,), jnp.float32),
            pltpu.VMEM_SHARED((_NPAD,), jnp.float32),
            pltpu.SemaphoreType.DMA,
            pltpu.SemaphoreType.DMA,
        ],
    )(ei_p, v_p, wb)


def _mlp_a_body(xt_ref, w1t_ref, wb_ref, h1p_ref, base_s):
    i = pl.program_id(0)

    @pl.when(i == 0)
    def _build():
        grow = lax.broadcasted_iota(jnp.int32, (_NIN, 128), 0)
        xt = xt_ref[...]
        w = wb_ref[0, 0]
        gb = wb_ref[0, 1]
        base_s[...] = jnp.where(grow < _NEQ,
                                jnp.maximum(w * xt + gb, 0.0), xt)

    h1p_ref[...] = lax.dot_general(
        w1t_ref[...], base_s[...], (((0,), (0,)), ((), ())),
        preferred_element_type=jnp.float32)


def _mlp_a(xt, W1t, wb2):
    return pl.pallas_call(
        _mlp_a_body,
        grid=(_NFB,),
        in_specs=[
            pl.BlockSpec((_NIN, 128), lambda i: (0, 0)),
            pl.BlockSpec((_NIN, _FB), lambda i: (0, i)),
            pl.BlockSpec((1, 2), lambda i: (0, 0)),
        ],
        out_specs=pl.BlockSpec((_FB, 128), lambda i: (i, 0)),
        out_shape=jax.ShapeDtypeStruct((512, 128), jnp.float32),
        scratch_shapes=[pltpu.VMEM((_NIN, 128), jnp.float32)],
    )(xt, W1t, wb2)


def _mlp_b_body(h1p_ref, xa_ref, os_ref, w1a_ref, w2_ref, wout_ref,
                b1_ref, b2_ref, bout_ref, wb_ref, mu_ref, lamb_ref):
    w = wb_ref[0, 0]
    gb = wb_ref[0, 1]
    grow = lax.broadcasted_iota(jnp.int32, (128, 128), 0)
    gcol = lax.broadcasted_iota(jnp.int32, (128, 128), 1)
    flat = grow * 128 + gcol
    base = jnp.maximum(w * xa_ref[...] + gb, 0.0)
    delta = jnp.where(flat < _NEQ, os_ref[...] - base, 0.0)
    corr = lax.dot_general(w1a_ref[...], delta, (((0,), (0,)), ((), ())),
                           preferred_element_type=jnp.float32)
    h1 = jnp.maximum(h1p_ref[...] + corr + b1_ref[...], 0.0)
    h2 = jnp.maximum(
        lax.dot_general(w2_ref[...], h1, (((1,), (0,)), ((), ())),
                        preferred_element_type=jnp.float32) + b2_ref[...],
        0.0)
    o = lax.dot_general(wout_ref[...], h2, (((1,), (0,)), ((), ())),
                        preferred_element_type=jnp.float32) + bout_ref[...]
    mu_ref[...] = o[:_MU, :]
    lamb_ref[...] = o[_MU:, :]


def _mlp_b(h1p, xa, os_sq, w1a, W2, Wout, b1, b2, bout, wb2):
    return pl.pallas_call(
        _mlp_b_body,
        out_shape=[jax.ShapeDtypeStruct((_MU, 128), jnp.float32),
                   jax.ShapeDtypeStruct((_NEQ, 128), jnp.float32)],
    )(h1p, xa, os_sq, w1a, W2, Wout, b1, b2, bout, wb2)


def kernel(x, edge_index, gcn_W, gcn_b, W1, b1, W2, b2, Wout, bout):
    w = gcn_W[0, 0]

    xt = x.T
    W1t = W1.T
    v_p = xt[: _NPAD // 128].reshape(-1)
    ei_p = jnp.pad(edge_index, ((0, 0), (0, _EPAD - _E)),
                   constant_values=_PAD_SLOT).reshape(2, _EPAD // _ECH, _ECH)
    wb = jnp.concatenate([jnp.full((16,), w, jnp.float32),
                          jnp.full((16,), gcn_b[0], jnp.float32)])
    wb2 = jnp.stack([w, gcn_b[0]]).reshape(1, 2)

    out_small = _gcn_sc(ei_p, v_p, wb)
    h1p = _mlp_a(xt, W1t, wb2)

    os_sq = jnp.pad(out_small.reshape(_NPAD // 128, 128), ((0, 48), (0, 0)))

    mu_t, lamb_t = _mlp_b(h1p, xt[:128], os_sq, W1t[:128], W2, Wout,
                          b1.reshape(512, 1), b2.reshape(256, 1),
                          bout.reshape(_NIN, 1), wb2)

    return (mu_t.T, lamb_t.T)


# speedup vs baseline: 24460.6647x; 1.0029x over previous
import jax
import jax.numpy as jnp
from jax import lax
from jax.experimental import pallas as pl
from jax.experimental.pallas import tpu as pltpu
from jax.experimental.pallas import tpu_sc as plsc

_NEQ = 10000
_MU = 500
_NPAD = 10240
_NPT = _NPAD // 16
_E = 160000
_EPAD = 163840
_ECH = 128
_CHUNKS = _EPAD // 16 // _ECH
_WAVE = 8
_PAD_SLOT = 10016
_OPAD = 6144

_NIN = 10500
_FB = 128
_NFB = 512 // _FB


def _rsqrt16(x):
    y = 1.0 / x
    for _ in range(24):
        y = y * (1.5 - 0.5 * x * y * y)
    return y


def _gcn_sc_body(ei_hbm, v_hbm, wb_hbm, out_hbm,
                 ridx, cidx, msg, ones, wb_v,
                 cnt_loc, v_loc, g_loc, dinv_loc, s_loc, out_loc,
                 sh_cnt, sh_g, sh_s, sem, sem2):
    c = lax.axis_index("c")
    sid = lax.axis_index("s")

    @pl.when(c == 0)
    def _body():
        nb = sid * _NPT
        eb = sid * _CHUNKS

        def z_body(k, carry):
            out_loc[pl.ds(k * 16, 16)] = jnp.zeros((16,), jnp.float32)
            return carry
        lax.fori_loop(0, _NPT // 16, z_body, None)
        for k in range(_ECH // 16):
            ones[pl.ds(k * 16, 16)] = jnp.ones((16,), jnp.float32)
        pltpu.sync_copy(out_loc, sh_cnt.at[pl.ds(nb, _NPT)])
        pltpu.sync_copy(out_loc, sh_s.at[pl.ds(nb, _NPT)])
        pltpu.sync_copy(out_loc.at[pl.ds(0, _OPAD // 16)],
                        out_hbm.at[pl.ds(_NPAD + sid * (_OPAD // 16),
                                         _OPAD // 16)])
        pltpu.sync_copy(wb_hbm, wb_v)
        pltpu.sync_copy(ei_hbm.at[0, pl.ds(eb, _CHUNKS)], ridx)
        pltpu.sync_copy(ei_hbm.at[1, pl.ds(eb, _CHUNKS)], cidx)
        plsc.subcore_barrier()

        def cnt_wave(wv, carry):
            for b in range(_WAVE):
                j = wv * _WAVE + b
                pltpu.async_copy(ones, sh_cnt.at[cidx.at[j]], sem, add=True)
            for b in range(_WAVE):
                j = wv * _WAVE + b
                pltpu.make_async_copy(ones, sh_cnt.at[cidx.at[j]], sem).wait()
            return carry
        lax.fori_loop(0, _CHUNKS // _WAVE, cnt_wave, None)
        plsc.subcore_barrier()

        pltpu.sync_copy(sh_cnt.at[pl.ds(nb, _NPT)], cnt_loc)
        pltpu.sync_copy(v_hbm.at[pl.ds(nb, _NPT)], v_loc)
        wvec = wb_v[pl.ds(0, 16)]

        def ew_body(k, carry):
            cnt16 = cnt_loc[pl.ds(k * 16, 16)]
            v16 = v_loc[pl.ds(k * 16, 16)]
            deg = cnt16 * 128.0 + 1.0
            y = _rsqrt16(deg)
            dinv_loc[pl.ds(k * 16, 16)] = y
            g_loc[pl.ds(k * 16, 16)] = y * v16 * wvec
            return carry
        lax.fori_loop(0, _NPT // 16, ew_body, None)
        pltpu.sync_copy(g_loc, sh_g.at[pl.ds(nb, _NPT)])
        plsc.subcore_barrier()

        def msg_wave(wv, carry):
            for b in range(_WAVE):
                j = wv * _WAVE + b
                pltpu.async_copy(sh_g.at[ridx.at[j]], msg.at[j], sem2)
            for b in range(_WAVE):
                j = wv * _WAVE + b
                pltpu.make_async_copy(sh_g.at[ridx.at[j]], msg.at[j], sem2).wait()
            for b in range(_WAVE):
                j = wv * _WAVE + b
                pltpu.async_copy(msg.at[j], sh_s.at[cidx.at[j]], sem, add=True)
            for b in range(_WAVE):
                j = wv * _WAVE + b
                pltpu.make_async_copy(msg.at[j], sh_s.at[cidx.at[j]], sem).wait()
            return carry
        lax.fori_loop(0, _CHUNKS // _WAVE, msg_wave, None)
        plsc.subcore_barrier()

        pltpu.sync_copy(sh_s.at[pl.ds(nb, _NPT)], s_loc)
        bvec = wb_v[pl.ds(16, 16)]

        def out_body(k, carry):
            s16 = s_loc[pl.ds(k * 16, 16)]
            d16 = dinv_loc[pl.ds(k * 16, 16)]
            v16 = v_loc[pl.ds(k * 16, 16)]
            o = 128.0 * d16 * s16 + d16 * d16 * v16 * wvec + bvec
            out_loc[pl.ds(k * 16, 16)] = jnp.maximum(o, 0.0)
            return carry
        lax.fori_loop(0, _NPT // 16, out_body, None)
        pltpu.sync_copy(out_loc, out_hbm.at[pl.ds(nb, _NPT)])


def _gcn_sc(ei_p, v_p, wb):
    mesh = plsc.VectorSubcoreMesh(core_axis_name="c", subcore_axis_name="s")
    return pl.kernel(
        _gcn_sc_body,
        out_type=jax.ShapeDtypeStruct((_NPAD + _OPAD,), jnp.float32),
        mesh=mesh,
        scratch_types=[
            pltpu.VMEM((_CHUNKS, _ECH), jnp.int32),
            pltpu.VMEM((_CHUNKS, _ECH), jnp.int32),
            pltpu.VMEM((_CHUNKS, _ECH), jnp.float32),
            pltpu.VMEM((_ECH,), jnp.float32),
            pltpu.VMEM((32,), jnp.float32),
            pltpu.VMEM((_NPT,), jnp.float32),
            pltpu.VMEM((_NPT,), jnp.float32),
            pltpu.VMEM((_NPT,), jnp.float32),
            pltpu.VMEM((_NPT,), jnp.float32),
            pltpu.VMEM((_NPT,), jnp.float32),
            pltpu.VMEM((_NPT,), jnp.float32),
            pltpu.VMEM_SHARED((_NPAD,), jnp.float32),
            pltpu.VMEM_SHARED((_NPAD,), jnp.float32),
            pltpu.VMEM_SHARED((_NPAD,), jnp.float32),
            pltpu.SemaphoreType.DMA,
            pltpu.SemaphoreType.DMA,
        ],
    )(ei_p, v_p, wb)


def _mlp_a_body(xt_ref, w1t_ref, wb_ref, h1p_ref, base_s):
    i = pl.program_id(0)

    @pl.when(i == 0)
    def _build():
        grow = lax.broadcasted_iota(jnp.int32, (_NIN, 128), 0)
        xt = xt_ref[...]
        w = wb_ref[0, 0]
        gb = wb_ref[0, 1]
        base_s[...] = jnp.where(grow < _NEQ,
                                jnp.maximum(w * xt + gb, 0.0), xt)

    h1p_ref[...] = lax.dot_general(
        base_s[...], w1t_ref[...], (((0,), (0,)), ((), ())),
        preferred_element_type=jnp.float32)


def _mlp_a(xt, W1t, wb2):
    return pl.pallas_call(
        _mlp_a_body,
        grid=(_NFB,),
        in_specs=[
            pl.BlockSpec((_NIN, 128), lambda i: (0, 0)),
            pl.BlockSpec((_NIN, _FB), lambda i: (0, i)),
            pl.BlockSpec((1, 2), lambda i: (0, 0)),
        ],
        out_specs=pl.BlockSpec((128, _FB), lambda i: (0, i)),
        out_shape=jax.ShapeDtypeStruct((128, 512), jnp.float32),
        scratch_shapes=[pltpu.VMEM((_NIN, 128), jnp.float32)],
    )(xt, W1t, wb2)


def _mlp_b_body(h1p_ref, xa_ref, os_ref, w1a_ref, w2_ref, wout_ref,
                b1_ref, b2_ref, bout_ref, wb_ref, mu_ref, lamb_ref):
    w = wb_ref[0, 0]
    gb = wb_ref[0, 1]
    grow = lax.broadcasted_iota(jnp.int32, (128, 128), 0)
    gcol = lax.broadcasted_iota(jnp.int32, (128, 128), 1)
    flat = grow * 128 + gcol
    base = jnp.maximum(w * xa_ref[...] + gb, 0.0)
    delta = jnp.where(flat < _NEQ, os_ref[...] - base, 0.0)
    corr = lax.dot_general(delta, w1a_ref[...], (((0,), (0,)), ((), ())),
                           preferred_element_type=jnp.float32)
    h1 = jnp.maximum(h1p_ref[...] + corr + b1_ref[...], 0.0)
    h2 = jnp.maximum(
        lax.dot_general(h1, w2_ref[...], (((1,), (1,)), ((), ())),
                        preferred_element_type=jnp.float32) + b2_ref[...],
        0.0)
    o = lax.dot_general(wout_ref[...], h2, (((1,), (1,)), ((), ())),
                        preferred_element_type=jnp.float32)
    o = o + lax.dot_general(bout_ref[...], jnp.ones((1, 128), jnp.float32),
                            (((0,), (0,)), ((), ())),
                            preferred_element_type=jnp.float32)
    mu_ref[...] = o[:_MU, :]
    lamb_ref[...] = o[_MU:, :]


def _mlp_b(h1p, xt, os_sq, W1t, W2, Wout, b1, b2, bout, wb2):
    return pl.pallas_call(
        _mlp_b_body,
        grid=(1,),
        in_specs=[
            pl.BlockSpec((128, 512), lambda i: (0, 0)),
            pl.BlockSpec((128, 128), lambda i: (0, 0)),
            pl.BlockSpec((128, 128), lambda i: (0, 0)),
            pl.BlockSpec((128, 512), lambda i: (0, 0)),
            pl.BlockSpec((256, 512), lambda i: (0, 0)),
            pl.BlockSpec((_NIN, 256), lambda i: (0, 0)),
            pl.BlockSpec((1, 512), lambda i: (0, 0)),
            pl.BlockSpec((1, 256), lambda i: (0, 0)),
            pl.BlockSpec((1, _NIN), lambda i: (0, 0)),
            pl.BlockSpec((1, 2), lambda i: (0, 0)),
        ],
        out_specs=[pl.BlockSpec((_MU, 128), lambda i: (0, 0)),
                   pl.BlockSpec((_NEQ, 128), lambda i: (0, 0))],
        out_shape=[jax.ShapeDtypeStruct((_MU, 128), jnp.float32),
                   jax.ShapeDtypeStruct((_NEQ, 128), jnp.float32)],
    )(h1p, xt, os_sq, W1t, W2, Wout, b1, b2, bout, wb2)


def kernel(x, edge_index, gcn_W, gcn_b, W1, b1, W2, b2, Wout, bout):
    w = gcn_W[0, 0]

    xt = x.T
    W1t = W1.T
    v_p = xt[: _NPAD // 128].reshape(-1)
    ei_p = jnp.pad(edge_index, ((0, 0), (0, _EPAD - _E)),
                   constant_values=_PAD_SLOT).reshape(2, _EPAD // _ECH, _ECH)
    wb = jnp.concatenate([jnp.full((16,), w, jnp.float32),
                          jnp.full((16,), gcn_b[0], jnp.float32)])
    wb2 = jnp.stack([w, gcn_b[0]]).reshape(1, 2)

    out_small = _gcn_sc(ei_p, v_p, wb)
    h1p = _mlp_a(xt, W1t, wb2)

    os_sq = out_small.reshape(128, 128)

    mu_t, lamb_t = _mlp_b(h1p, xt, os_sq, W1t, W2, Wout,
                          b1.reshape(1, 512), b2.reshape(1, 256),
                          bout.reshape(1, _NIN), wb2)

    return (mu_t.T, lamb_t.T)


# speedup vs baseline: 25597.8335x; 1.0465x over previous
import jax
import jax.numpy as jnp
from jax import lax
from jax.experimental import pallas as pl
from jax.experimental.pallas import tpu as pltpu
from jax.experimental.pallas import tpu_sc as plsc

_NEQ = 10000
_MU = 500
_NPAD = 10240
_NPT = _NPAD // 16
_E = 160000
_EPAD = 163840
_ECH = 128
_CHUNKS = _EPAD // 16 // _ECH
_MCHUNKS = _EPAD // 32 // _ECH
_WAVE = 8
_PAD_SLOT = 10016
_OPAD = 6144

_NIN = 10500
_FB = 128
_NFB = 512 // _FB


def _rsqrt16(x):
    y = 1.0 / x
    for _ in range(24):
        y = y * (1.5 - 0.5 * x * y * y)
    return y


def _gcn_sc_body(ei_hbm, v_hbm, wb_hbm, out0_hbm, out1_hbm,
                 ridx, cidx, msg, ones, wb_v,
                 cnt_loc, v_loc, g_loc, dinv_loc, s_loc, out_loc,
                 sh_cnt, sh_g, sh_s, sem, sem2):
    c = lax.axis_index("c")
    sid = lax.axis_index("s")

    nb = sid * _NPT
    eb = sid * _CHUNKS
    ebm = (c * 16 + sid) * _MCHUNKS

    def z_body(k, carry):
        out_loc[pl.ds(k * 16, 16)] = jnp.zeros((16,), jnp.float32)
        return carry
    lax.fori_loop(0, _NPT // 16, z_body, None)
    for k in range(_ECH // 16):
        ones[pl.ds(k * 16, 16)] = jnp.ones((16,), jnp.float32)
    pltpu.sync_copy(out_loc, sh_cnt.at[pl.ds(nb, _NPT)])
    pltpu.sync_copy(out_loc, sh_s.at[pl.ds(nb, _NPT)])

    @pl.when(c == 0)
    def _z0():
        pltpu.sync_copy(out_loc.at[pl.ds(0, _OPAD // 16)],
                        out0_hbm.at[pl.ds(_NPAD + sid * (_OPAD // 16),
                                          _OPAD // 16)])

    @pl.when(c == 1)
    def _z1():
        pltpu.sync_copy(out_loc.at[pl.ds(0, _OPAD // 16)],
                        out1_hbm.at[pl.ds(_NPAD + sid * (_OPAD // 16),
                                          _OPAD // 16)])

    pltpu.sync_copy(wb_hbm, wb_v)
    pltpu.sync_copy(ei_hbm.at[1, pl.ds(eb, _CHUNKS)],
                    cidx.at[pl.ds(0, _CHUNKS)])
    pltpu.sync_copy(ei_hbm.at[0, pl.ds(ebm, _MCHUNKS)], ridx)
    pltpu.sync_copy(ei_hbm.at[1, pl.ds(ebm, _MCHUNKS)],
                    cidx.at[pl.ds(_CHUNKS, _MCHUNKS)])
    plsc.subcore_barrier()

    def cnt_wave(wv, carry):
        for b in range(_WAVE):
            j = wv * _WAVE + b
            pltpu.async_copy(ones, sh_cnt.at[cidx.at[j]], sem, add=True)
        for b in range(_WAVE):
            j = wv * _WAVE + b
            pltpu.make_async_copy(ones, sh_cnt.at[cidx.at[j]], sem).wait()
        return carry
    lax.fori_loop(0, _CHUNKS // _WAVE, cnt_wave, None)
    plsc.subcore_barrier()

    pltpu.sync_copy(sh_cnt.at[pl.ds(nb, _NPT)], cnt_loc)
    pltpu.sync_copy(v_hbm.at[pl.ds(nb, _NPT)], v_loc)
    wvec = wb_v[pl.ds(0, 16)]

    def ew_body(k, carry):
        cnt16 = cnt_loc[pl.ds(k * 16, 16)]
        v16 = v_loc[pl.ds(k * 16, 16)]
        deg = cnt16 * 128.0 + 1.0
        y = _rsqrt16(deg)
        dinv_loc[pl.ds(k * 16, 16)] = y
        g_loc[pl.ds(k * 16, 16)] = y * v16 * wvec
        return carry
    lax.fori_loop(0, _NPT // 16, ew_body, None)
    pltpu.sync_copy(g_loc, sh_g.at[pl.ds(nb, _NPT)])
    plsc.subcore_barrier()

    def msg_wave(wv, carry):
        for b in range(_WAVE):
            j = wv * _WAVE + b
            pltpu.async_copy(sh_g.at[ridx.at[j]], msg.at[j], sem2)
        for b in range(_WAVE):
            j = wv * _WAVE + b
            pltpu.make_async_copy(sh_g.at[ridx.at[j]], msg.at[j], sem2).wait()
        for b in range(_WAVE):
            j = wv * _WAVE + b
            pltpu.async_copy(msg.at[j], sh_s.at[cidx.at[_CHUNKS + j]],
                             sem, add=True)
        for b in range(_WAVE):
            j = wv * _WAVE + b
            pltpu.make_async_copy(msg.at[j], sh_s.at[cidx.at[_CHUNKS + j]],
                                  sem).wait()
        return carry
    lax.fori_loop(0, _MCHUNKS // _WAVE, msg_wave, None)
    plsc.subcore_barrier()

    pltpu.sync_copy(sh_s.at[pl.ds(nb, _NPT)], s_loc)
    bvec = wb_v[pl.ds(16, 16)]

    def out_body0(k, carry):
        s16 = s_loc[pl.ds(k * 16, 16)]
        d16 = dinv_loc[pl.ds(k * 16, 16)]
        v16 = v_loc[pl.ds(k * 16, 16)]
        out_loc[pl.ds(k * 16, 16)] = (128.0 * d16 * s16
                                      + d16 * d16 * v16 * wvec + bvec)
        return carry

    def out_body1(k, carry):
        s16 = s_loc[pl.ds(k * 16, 16)]
        d16 = dinv_loc[pl.ds(k * 16, 16)]
        out_loc[pl.ds(k * 16, 16)] = 128.0 * d16 * s16
        return carry

    @pl.when(c == 0)
    def _w0():
        lax.fori_loop(0, _NPT // 16, out_body0, None)
        pltpu.sync_copy(out_loc, out0_hbm.at[pl.ds(nb, _NPT)])

    @pl.when(c == 1)
    def _w1():
        lax.fori_loop(0, _NPT // 16, out_body1, None)
        pltpu.sync_copy(out_loc, out1_hbm.at[pl.ds(nb, _NPT)])


def _gcn_sc(ei_p, v_p, wb):
    mesh = plsc.VectorSubcoreMesh(core_axis_name="c", subcore_axis_name="s")
    return pl.kernel(
        _gcn_sc_body,
        out_type=[jax.ShapeDtypeStruct((_NPAD + _OPAD,), jnp.float32),
                  jax.ShapeDtypeStruct((_NPAD + _OPAD,), jnp.float32)],
        mesh=mesh,
        scratch_types=[
            pltpu.VMEM((_MCHUNKS, _ECH), jnp.int32),
            pltpu.VMEM((_CHUNKS + _MCHUNKS, _ECH), jnp.int32),
            pltpu.VMEM((_MCHUNKS, _ECH), jnp.float32),
            pltpu.VMEM((_ECH,), jnp.float32),
            pltpu.VMEM((32,), jnp.float32),
            pltpu.VMEM((_NPT,), jnp.float32),
            pltpu.VMEM((_NPT,), jnp.float32),
            pltpu.VMEM((_NPT,), jnp.float32),
            pltpu.VMEM((_NPT,), jnp.float32),
            pltpu.VMEM((_NPT,), jnp.float32),
            pltpu.VMEM((_NPT,), jnp.float32),
            pltpu.VMEM_SHARED((_NPAD,), jnp.float32),
            pltpu.VMEM_SHARED((_NPAD,), jnp.float32),
            pltpu.VMEM_SHARED((_NPAD,), jnp.float32),
            pltpu.SemaphoreType.DMA,
            pltpu.SemaphoreType.DMA,
        ],
    )(ei_p, v_p, wb)


def _mlp_a_body(xt_ref, w1t_ref, wb_ref, h1p_ref, base_s):
    i = pl.program_id(0)

    @pl.when(i == 0)
    def _build():
        grow = lax.broadcasted_iota(jnp.int32, (_NIN, 128), 0)
        xt = xt_ref[...]
        w = wb_ref[0, 0]
        gb = wb_ref[0, 1]
        base_s[...] = jnp.where(grow < _NEQ,
                                jnp.maximum(w * xt + gb, 0.0), xt)

    h1p_ref[...] = lax.dot_general(
        base_s[...], w1t_ref[...], (((0,), (0,)), ((), ())),
        preferred_element_type=jnp.float32)


def _mlp_a(xt, W1t, wb2):
    return pl.pallas_call(
        _mlp_a_body,
        grid=(_NFB,),
        in_specs=[
            pl.BlockSpec((_NIN, 128), lambda i: (0, 0)),
            pl.BlockSpec((_NIN, _FB), lambda i: (0, i)),
            pl.BlockSpec((1, 2), lambda i: (0, 0)),
        ],
        out_specs=pl.BlockSpec((128, _FB), lambda i: (0, i)),
        out_shape=jax.ShapeDtypeStruct((128, 512), jnp.float32),
        scratch_shapes=[pltpu.VMEM((_NIN, 128), jnp.float32)],
    )(xt, W1t, wb2)


def _mlp_b_body(h1p_ref, xa_ref, os0_ref, os1_ref, w1a_ref, w2_ref, wout_ref,
                b1_ref, b2_ref, bout_ref, wb_ref, mu_ref, lamb_ref):
    w = wb_ref[0, 0]
    gb = wb_ref[0, 1]
    grow = lax.broadcasted_iota(jnp.int32, (128, 128), 0)
    gcol = lax.broadcasted_iota(jnp.int32, (128, 128), 1)
    flat = grow * 128 + gcol
    base = jnp.maximum(w * xa_ref[...] + gb, 0.0)
    sub = jnp.maximum(os0_ref[...] + os1_ref[...], 0.0)
    delta = jnp.where(flat < _NEQ, sub - base, 0.0)
    corr = lax.dot_general(delta, w1a_ref[...], (((0,), (0,)), ((), ())),
                           preferred_element_type=jnp.float32)
    h1 = jnp.maximum(h1p_ref[...] + corr + b1_ref[...], 0.0)
    h2 = jnp.maximum(
        lax.dot_general(h1, w2_ref[...], (((1,), (1,)), ((), ())),
                        preferred_element_type=jnp.float32) + b2_ref[...],
        0.0)
    o = lax.dot_general(wout_ref[...], h2, (((1,), (1,)), ((), ())),
                        preferred_element_type=jnp.float32)
    o = o + lax.dot_general(bout_ref[...], jnp.ones((1, 128), jnp.float32),
                            (((0,), (0,)), ((), ())),
                            preferred_element_type=jnp.float32)
    mu_ref[...] = o[:_MU, :]
    lamb_ref[...] = o[_MU:, :]


def _mlp_b(h1p, xt, os0, os1, W1t, W2, Wout, b1, b2, bout, wb2):
    return pl.pallas_call(
        _mlp_b_body,
        grid=(1,),
        in_specs=[
            pl.BlockSpec((128, 512), lambda i: (0, 0)),
            pl.BlockSpec((128, 128), lambda i: (0, 0)),
            pl.BlockSpec((128, 128), lambda i: (0, 0)),
            pl.BlockSpec((128, 128), lambda i: (0, 0)),
            pl.BlockSpec((128, 512), lambda i: (0, 0)),
            pl.BlockSpec((256, 512), lambda i: (0, 0)),
            pl.BlockSpec((_NIN, 256), lambda i: (0, 0)),
            pl.BlockSpec((1, 512), lambda i: (0, 0)),
            pl.BlockSpec((1, 256), lambda i: (0, 0)),
            pl.BlockSpec((1, _NIN), lambda i: (0, 0)),
            pl.BlockSpec((1, 2), lambda i: (0, 0)),
        ],
        out_specs=[pl.BlockSpec((_MU, 128), lambda i: (0, 0)),
                   pl.BlockSpec((_NEQ, 128), lambda i: (0, 0))],
        out_shape=[jax.ShapeDtypeStruct((_MU, 128), jnp.float32),
                   jax.ShapeDtypeStruct((_NEQ, 128), jnp.float32)],
    )(h1p, xt, os0, os1, W1t, W2, Wout, b1, b2, bout, wb2)


def kernel(x, edge_index, gcn_W, gcn_b, W1, b1, W2, b2, Wout, bout):
    w = gcn_W[0, 0]

    xt = x.T
    W1t = W1.T
    v_p = xt[: _NPAD // 128].reshape(-1)
    ei_p = jnp.pad(edge_index, ((0, 0), (0, _EPAD - _E)),
                   constant_values=_PAD_SLOT).reshape(2, _EPAD // _ECH, _ECH)
    wb = jnp.concatenate([jnp.full((16,), w, jnp.float32),
                          jnp.full((16,), gcn_b[0], jnp.float32)])
    wb2 = jnp.stack([w, gcn_b[0]]).reshape(1, 2)

    p0, p1 = _gcn_sc(ei_p, v_p, wb)
    h1p = _mlp_a(xt, W1t, wb2)

    os0 = p0.reshape(128, 128)
    os1 = p1.reshape(128, 128)

    mu_t, lamb_t = _mlp_b(h1p, xt, os0, os1, W1t, W2, Wout,
                          b1.reshape(1, 512), b2.reshape(1, 256),
                          bout.reshape(1, _NIN), wb2)

    return (mu_t.T, lamb_t.T)


# speedup vs baseline: 25754.8867x; 1.0061x over previous
import jax
import jax.numpy as jnp
from jax import lax
from jax.experimental import pallas as pl
from jax.experimental.pallas import tpu as pltpu
from jax.experimental.pallas import tpu_sc as plsc

_NEQ = 10000
_MU = 500
_NPAD = 10240
_NPT = _NPAD // 16
_E = 160000
_EPAD = 163840
_ECH = 128
_CHUNKS = _EPAD // 16 // _ECH
_MCHUNKS = _EPAD // 32 // _ECH
_WAVE = 20
_PAD_SLOT = 10016
_OPAD = 6144

_NIN = 10500
_FB = 128
_NFB = 512 // _FB


def _rsqrt16(x):
    y = 1.0 / x
    for _ in range(24):
        y = y * (1.5 - 0.5 * x * y * y)
    return y


def _gcn_sc_body(ei_hbm, v_hbm, wb_hbm, out0_hbm, out1_hbm,
                 ridx, cidx, msg, ones, wb_v,
                 cnt_loc, v_loc, g_loc, dinv_loc, s_loc, out_loc,
                 sh_cnt, sh_g, sh_s, sem, sem2):
    c = lax.axis_index("c")
    sid = lax.axis_index("s")

    nb = sid * _NPT
    eb = sid * _CHUNKS
    ebm = (c * 16 + sid) * _MCHUNKS

    def z_body(k, carry):
        out_loc[pl.ds(k * 16, 16)] = jnp.zeros((16,), jnp.float32)
        return carry
    lax.fori_loop(0, _NPT // 16, z_body, None)
    for k in range(_ECH // 16):
        ones[pl.ds(k * 16, 16)] = jnp.ones((16,), jnp.float32)
    pltpu.sync_copy(out_loc, sh_cnt.at[pl.ds(nb, _NPT)])
    pltpu.sync_copy(out_loc, sh_s.at[pl.ds(nb, _NPT)])

    @pl.when(c == 0)
    def _z0():
        pltpu.sync_copy(out_loc.at[pl.ds(0, _OPAD // 16)],
                        out0_hbm.at[pl.ds(_NPAD + sid * (_OPAD // 16),
                                          _OPAD // 16)])

    @pl.when(c == 1)
    def _z1():
        pltpu.sync_copy(out_loc.at[pl.ds(0, _OPAD // 16)],
                        out1_hbm.at[pl.ds(_NPAD + sid * (_OPAD // 16),
                                          _OPAD // 16)])

    pltpu.sync_copy(wb_hbm, wb_v)
    pltpu.sync_copy(ei_hbm.at[1, pl.ds(eb, _CHUNKS)],
                    cidx.at[pl.ds(0, _CHUNKS)])
    pltpu.sync_copy(ei_hbm.at[0, pl.ds(ebm, _MCHUNKS)], ridx)
    pltpu.sync_copy(ei_hbm.at[1, pl.ds(ebm, _MCHUNKS)],
                    cidx.at[pl.ds(_CHUNKS, _MCHUNKS)])
    plsc.subcore_barrier()

    def cnt_wave(wv, carry):
        for b in range(_WAVE):
            j = wv * _WAVE + b
            pltpu.async_copy(ones, sh_cnt.at[cidx.at[j]], sem, add=True)
        for b in range(_WAVE):
            j = wv * _WAVE + b
            pltpu.make_async_copy(ones, sh_cnt.at[cidx.at[j]], sem).wait()
        return carry
    lax.fori_loop(0, _CHUNKS // _WAVE, cnt_wave, None)
    plsc.subcore_barrier()

    pltpu.sync_copy(sh_cnt.at[pl.ds(nb, _NPT)], cnt_loc)
    pltpu.sync_copy(v_hbm.at[pl.ds(nb, _NPT)], v_loc)
    wvec = wb_v[pl.ds(0, 16)]

    def ew_body(k, carry):
        cnt16 = cnt_loc[pl.ds(k * 16, 16)]
        v16 = v_loc[pl.ds(k * 16, 16)]
        deg = cnt16 * 128.0 + 1.0
        y = _rsqrt16(deg)
        dinv_loc[pl.ds(k * 16, 16)] = y
        g_loc[pl.ds(k * 16, 16)] = y * v16 * wvec
        return carry
    lax.fori_loop(0, _NPT // 16, ew_body, None)
    pltpu.sync_copy(g_loc, sh_g.at[pl.ds(nb, _NPT)])
    plsc.subcore_barrier()

    def msg_wave(wv, carry):
        for b in range(_WAVE):
            j = wv * _WAVE + b
            pltpu.async_copy(sh_g.at[ridx.at[j]], msg.at[j], sem2)
        for b in range(_WAVE):
            j = wv * _WAVE + b
            pltpu.make_async_copy(sh_g.at[ridx.at[j]], msg.at[j], sem2).wait()
        for b in range(_WAVE):
            j = wv * _WAVE + b
            pltpu.async_copy(msg.at[j], sh_s.at[cidx.at[_CHUNKS + j]],
                             sem, add=True)
        for b in range(_WAVE):
            j = wv * _WAVE + b
            pltpu.make_async_copy(msg.at[j], sh_s.at[cidx.at[_CHUNKS + j]],
                                  sem).wait()
        return carry
    lax.fori_loop(0, _MCHUNKS // _WAVE, msg_wave, None)
    plsc.subcore_barrier()

    pltpu.sync_copy(sh_s.at[pl.ds(nb, _NPT)], s_loc)
    bvec = wb_v[pl.ds(16, 16)]

    def out_body0(k, carry):
        s16 = s_loc[pl.ds(k * 16, 16)]
        d16 = dinv_loc[pl.ds(k * 16, 16)]
        v16 = v_loc[pl.ds(k * 16, 16)]
        out_loc[pl.ds(k * 16, 16)] = (128.0 * d16 * s16
                                      + d16 * d16 * v16 * wvec + bvec)
        return carry

    def out_body1(k, carry):
        s16 = s_loc[pl.ds(k * 16, 16)]
        d16 = dinv_loc[pl.ds(k * 16, 16)]
        out_loc[pl.ds(k * 16, 16)] = 128.0 * d16 * s16
        return carry

    @pl.when(c == 0)
    def _w0():
        lax.fori_loop(0, _NPT // 16, out_body0, None)
        pltpu.sync_copy(out_loc, out0_hbm.at[pl.ds(nb, _NPT)])

    @pl.when(c == 1)
    def _w1():
        lax.fori_loop(0, _NPT // 16, out_body1, None)
        pltpu.sync_copy(out_loc, out1_hbm.at[pl.ds(nb, _NPT)])


def _gcn_sc(ei_p, v_p, wb):
    mesh = plsc.VectorSubcoreMesh(core_axis_name="c", subcore_axis_name="s")
    return pl.kernel(
        _gcn_sc_body,
        out_type=[jax.ShapeDtypeStruct((_NPAD + _OPAD,), jnp.float32),
                  jax.ShapeDtypeStruct((_NPAD + _OPAD,), jnp.float32)],
        mesh=mesh,
        scratch_types=[
            pltpu.VMEM((_MCHUNKS, _ECH), jnp.int32),
            pltpu.VMEM((_CHUNKS + _MCHUNKS, _ECH), jnp.int32),
            pltpu.VMEM((_MCHUNKS, _ECH), jnp.float32),
            pltpu.VMEM((_ECH,), jnp.float32),
            pltpu.VMEM((32,), jnp.float32),
            pltpu.VMEM((_NPT,), jnp.float32),
            pltpu.VMEM((_NPT,), jnp.float32),
            pltpu.VMEM((_NPT,), jnp.float32),
            pltpu.VMEM((_NPT,), jnp.float32),
            pltpu.VMEM((_NPT,), jnp.float32),
            pltpu.VMEM((_NPT,), jnp.float32),
            pltpu.VMEM_SHARED((_NPAD,), jnp.float32),
            pltpu.VMEM_SHARED((_NPAD,), jnp.float32),
            pltpu.VMEM_SHARED((_NPAD,), jnp.float32),
            pltpu.SemaphoreType.DMA,
            pltpu.SemaphoreType.DMA,
        ],
    )(ei_p, v_p, wb)


def _mlp_a_body(xt_ref, w1t_ref, wb_ref, h1p_ref, base_s):
    i = pl.program_id(0)

    @pl.when(i == 0)
    def _build():
        grow = lax.broadcasted_iota(jnp.int32, (_NIN, 128), 0)
        xt = xt_ref[...]
        w = wb_ref[0, 0]
        gb = wb_ref[0, 1]
        base_s[...] = jnp.where(grow < _NEQ,
                                jnp.maximum(w * xt + gb, 0.0), xt)

    h1p_ref[...] = lax.dot_general(
        base_s[...], w1t_ref[...], (((0,), (0,)), ((), ())),
        preferred_element_type=jnp.float32)


def _mlp_a(xt, W1t, wb2):
    return pl.pallas_call(
        _mlp_a_body,
        grid=(_NFB,),
        in_specs=[
            pl.BlockSpec((_NIN, 128), lambda i: (0, 0)),
            pl.BlockSpec((_NIN, _FB), lambda i: (0, i)),
            pl.BlockSpec((1, 2), lambda i: (0, 0)),
        ],
        out_specs=pl.BlockSpec((128, _FB), lambda i: (0, i)),
        out_shape=jax.ShapeDtypeStruct((128, 512), jnp.float32),
        scratch_shapes=[pltpu.VMEM((_NIN, 128), jnp.float32)],
    )(xt, W1t, wb2)


def _mlp_b_body(h1p_ref, xa_ref, os0_ref, os1_ref, w1a_ref, w2_ref, wout_ref,
                b1_ref, b2_ref, bout_ref, wb_ref, mu_ref, lamb_ref):
    w = wb_ref[0, 0]
    gb = wb_ref[0, 1]
    grow = lax.broadcasted_iota(jnp.int32, (128, 128), 0)
    gcol = lax.broadcasted_iota(jnp.int32, (128, 128), 1)
    flat = grow * 128 + gcol
    base = jnp.maximum(w * xa_ref[...] + gb, 0.0)
    sub = jnp.maximum(os0_ref[...] + os1_ref[...], 0.0)
    delta = jnp.where(flat < _NEQ, sub - base, 0.0)
    corr = lax.dot_general(delta, w1a_ref[...], (((0,), (0,)), ((), ())),
                           preferred_element_type=jnp.float32)
    h1 = jnp.maximum(h1p_ref[...] + corr + b1_ref[...], 0.0)
    h2 = jnp.maximum(
        lax.dot_general(h1, w2_ref[...], (((1,), (1,)), ((), ())),
                        preferred_element_type=jnp.float32) + b2_ref[...],
        0.0)
    o = lax.dot_general(wout_ref[...], h2, (((1,), (1,)), ((), ())),
                        preferred_element_type=jnp.float32)
    o = o + lax.dot_general(bout_ref[...], jnp.ones((1, 128), jnp.float32),
                            (((0,), (0,)), ((), ())),
                            preferred_element_type=jnp.float32)
    mu_ref[...] = o[:_MU, :]
    lamb_ref[...] = o[_MU:, :]


def _mlp_b(h1p, xt, os0, os1, W1t, W2, Wout, b1, b2, bout, wb2):
    return pl.pallas_call(
        _mlp_b_body,
        grid=(1,),
        in_specs=[
            pl.BlockSpec((128, 512), lambda i: (0, 0)),
            pl.BlockSpec((128, 128), lambda i: (0, 0)),
            pl.BlockSpec((128, 128), lambda i: (0, 0)),
            pl.BlockSpec((128, 128), lambda i: (0, 0)),
            pl.BlockSpec((128, 512), lambda i: (0, 0)),
            pl.BlockSpec((256, 512), lambda i: (0, 0)),
            pl.BlockSpec((_NIN, 256), lambda i: (0, 0)),
            pl.BlockSpec((1, 512), lambda i: (0, 0)),
            pl.BlockSpec((1, 256), lambda i: (0, 0)),
            pl.BlockSpec((1, _NIN), lambda i: (0, 0)),
            pl.BlockSpec((1, 2), lambda i: (0, 0)),
        ],
        out_specs=[pl.BlockSpec((_MU, 128), lambda i: (0, 0)),
                   pl.BlockSpec((_NEQ, 128), lambda i: (0, 0))],
        out_shape=[jax.ShapeDtypeStruct((_MU, 128), jnp.float32),
                   jax.ShapeDtypeStruct((_NEQ, 128), jnp.float32)],
    )(h1p, xt, os0, os1, W1t, W2, Wout, b1, b2, bout, wb2)


def kernel(x, edge_index, gcn_W, gcn_b, W1, b1, W2, b2, Wout, bout):
    w = gcn_W[0, 0]

    xt = x.T
    W1t = W1.T
    v_p = xt[: _NPAD // 128].reshape(-1)
    ei_p = jnp.pad(edge_index, ((0, 0), (0, _EPAD - _E)),
                   constant_values=_PAD_SLOT).reshape(2, _EPAD // _ECH, _ECH)
    wb = jnp.concatenate([jnp.full((16,), w, jnp.float32),
                          jnp.full((16,), gcn_b[0], jnp.float32)])
    wb2 = jnp.stack([w, gcn_b[0]]).reshape(1, 2)

    p0, p1 = _gcn_sc(ei_p, v_p, wb)
    h1p = _mlp_a(xt, W1t, wb2)

    os0 = p0.reshape(128, 128)
    os1 = p1.reshape(128, 128)

    mu_t, lamb_t = _mlp_b(h1p, xt, os0, os1, W1t, W2, Wout,
                          b1.reshape(1, 512), b2.reshape(1, 256),
                          bout.reshape(1, _NIN), wb2)

    return (mu_t.T, lamb_t.T)
